# baseline (speedup 1.0000x reference)
import functools

import jax
import jax.numpy as jnp
from jax import lax
from jax.experimental import pallas as pl
from jax.experimental.pallas import tpu as pltpu
from jax.experimental.pallas import tpu_sc as plsc

NC = 2
NS = 16
NW = NC * NS
LANES = 16
WAVE = 32
RB = 4


def _make_sc_partials(B, D):
    b_per_w = B // NW
    nwaves = b_per_w // WAVE
    mesh = plsc.VectorSubcoreMesh(core_axis_name="c", subcore_axis_name="s")

    @functools.partial(
        pl.kernel,
        mesh=mesh,
        compiler_params=pltpu.CompilerParams(needs_layout_passes=False),
        out_type=jax.ShapeDtypeStruct((NW, LANES), jnp.float32),
        scratch_types=[
            pltpu.SMEM((b_per_w,), jnp.int32),
            pltpu.VMEM_SHARED((NS, b_per_w), jnp.int32),
            pltpu.VMEM((b_per_w, D), jnp.float32),
            pltpu.VMEM((RB * WAVE, D), jnp.float32),
            pltpu.VMEM((LANES,), jnp.float32),
            pltpu.SemaphoreType.DMA,
            pltpu.SemaphoreType.DMA,
            pltpu.SemaphoreType.DMA,
        ],
    )
    def sc_partials(feats_hbm, tgt_hbm, table_hbm, out_hbm,
                    idx_s, idx_v, feats_v, rows_v, tot_v,
                    sem_f, sem_a, sem_b):
        wid = lax.axis_index("s") * NC + lax.axis_index("c")

        sid = lax.axis_index("s")
        pltpu.sync_copy(tgt_hbm.at[wid], idx_v.at[sid])
        pltpu.sync_copy(idx_v.at[sid], idx_s)
        cf = pltpu.async_copy(feats_hbm.at[wid], feats_v, sem_f)
        sems = [sem_a, sem_b]

        def fire(w, sem):
            slot0 = (w % RB) * WAVE

            def body(i, _):
                t = idx_s[w * WAVE + i]
                pltpu.async_copy(table_hbm.at[t], rows_v.at[slot0 + i], sem)
                return 0

            lax.fori_loop(0, WAVE, body, 0)

        def drain(w, sem):
            slot0 = (w % RB) * WAVE

            def body(i, _):
                pltpu.make_async_copy(
                    table_hbm.at[0], rows_v.at[slot0 + i], sem
                ).wait()
                return 0

            lax.fori_loop(0, WAVE, body, 0)

        fire(0, sems[0])
        fire(1, sems[1])
        cf.wait()

        iota = lax.iota(jnp.int32, 16)
        total = jnp.zeros((LANES,), jnp.float32)

        for w in range(nwaves):
            drain(w, sems[w % 2])
            if w + 2 < nwaves:
                fire(w + 2, sems[w % 2])
            for g2 in range(WAVE // LANES):
                r0 = w * WAVE + g2 * LANES
                rowvec = iota + r0
                slotvec = iota + ((w % RB) * WAVE + g2 * LANES)

                def col(d, dist, rowvec=rowvec, slotvec=slotvec):
                    dv = jnp.full((LANES,), d, jnp.int32)
                    fv = plsc.load_gather(feats_v, [rowvec, dv])
                    cv = plsc.load_gather(rows_v, [slotvec, dv])
                    df = fv - cv
                    return dist + df * df

                dist = lax.fori_loop(0, D, col, jnp.zeros((LANES,), jnp.float32))
                total = total + jnp.clip(dist, 1e-12, 1e12)

        tot_v[...] = total
        pltpu.sync_copy(tot_v, out_hbm.at[wid])

    return sc_partials


def kernel(feats, targets, centers):
    B, D = feats.shape
    b_per_w = B // NW

    feats_r = feats.reshape(NW, b_per_w, D)
    tgt_r = targets.astype(jnp.int32).reshape(NW, b_per_w)

    partials = _make_sc_partials(B, D)(feats_r, tgt_r, centers)

    def tc_reduce(p_ref, o_ref):
        s = 0.5 * jnp.sum(p_ref[...]) * (1.0 / B)
        o_ref[...] = jnp.broadcast_to(s, (1, 1))

    loss = pl.pallas_call(
        tc_reduce,
        out_shape=jax.ShapeDtypeStruct((1, 1), jnp.float32),
    )(partials)
    return loss[0, 0]
